# Initial kernel scaffold; baseline (speedup 1.0000x reference)
import jax
import jax.numpy as jnp
from jax.experimental import pallas as pl


def kernel(sequence_output, pair_indices, W, b):
    raise NotImplementedError("write your pallas kernel here")


# speedup vs baseline: 13.3956x; 13.3956x over previous
import jax
import jax.numpy as jnp
from jax.experimental import pallas as pl

B, S, H, P, NUM_LABELS = 8, 2048, 1024, 64, 2


def _pair_kernel(seq_ref, idx_ref, wt_ref, b_ref, out_ref):
    seq = seq_ref[0]
    proj = jnp.dot(seq, wt_ref[:], preferred_element_type=jnp.float32)
    idx = idx_ref[0]
    j = jax.lax.broadcasted_iota(jnp.int32, (P, S), 1)
    s1 = idx[:, 0:1]
    e1 = idx[:, 1:2]
    s2 = idx[:, 2:3]
    e2 = idx[:, 3:4]
    cnt1 = (e1 - s1).astype(jnp.float32)
    cnt2 = (e2 - s2).astype(jnp.float32)
    m1 = ((j >= s1) & (j < e1)).astype(jnp.float32) / cnt1
    m2 = ((j >= s2) & (j < e2)).astype(jnp.float32) / cnt2
    l1 = jnp.dot(m1, proj[:, 0:2], preferred_element_type=jnp.float32)
    l2 = jnp.dot(m2, proj[:, 2:4], preferred_element_type=jnp.float32)
    out_ref[0] = l1 + l2 + b_ref[:]


def kernel(sequence_output, pair_indices, W, b):
    Wt = jnp.concatenate([W[:, :H].T, W[:, H:].T], axis=1)
    logits = pl.pallas_call(
        _pair_kernel,
        grid=(B,),
        in_specs=[
            pl.BlockSpec((1, S, H), lambda i: (i, 0, 0)),
            pl.BlockSpec((1, P, 5), lambda i: (i, 0, 0)),
            pl.BlockSpec((H, 4), lambda i: (0, 0)),
            pl.BlockSpec((1, NUM_LABELS), lambda i: (0, 0)),
        ],
        out_specs=pl.BlockSpec((1, P, NUM_LABELS), lambda i: (i, 0, 0)),
        out_shape=jax.ShapeDtypeStruct((B, P, NUM_LABELS), jnp.float32),
    )(sequence_output, pair_indices, Wt, b.reshape(1, NUM_LABELS))
    labels = pair_indices[..., 4].reshape(-1)
    return (logits.reshape(-1, NUM_LABELS), labels)
